# initial kernel scaffold (unmeasured)
import jax
import jax.numpy as jnp
from jax import lax
from jax.experimental import pallas as pl
from jax.experimental.pallas import tpu as pltpu

N_Y = 4
S = 1024
D = 2048
H = 16
DH = 128
DR = 32
CHUNK = S // N_Y
KV = 2 * D
SCALE = (DH + DR) ** -0.5
BF = jnp.bfloat16


def _reduce_body(x_ref, wdkv_ref, wuk_ref, wuv_ref, wq_ref, wqr_ref, wkr_ref,
                 k_ref, v_ref, q_ref, qr_ref, kr_ref,
                 acc_ref, rs_recv_ref,
                 rs_send_sems, rs_recv_sems, ag_send_sems, ag_recv_sems):
    my_x = lax.axis_index("x")
    my_y = lax.axis_index("y")
    my_z = lax.axis_index("z")
    right = (my_x, (my_y + 1) % N_Y, my_z)
    left = (my_x, (my_y - 1) % N_Y, my_z)

    barrier = pltpu.get_barrier_semaphore()
    for nbr in (left, right):
        pl.semaphore_signal(barrier, inc=1, device_id=nbr,
                            device_id_type=pl.DeviceIdType.MESH)
    pl.semaphore_wait(barrier, 2)

    x = x_ref[...]
    c = jnp.dot(x, wdkv_ref[...], preferred_element_type=jnp.float32).astype(BF)
    acc_ref[:, :D] = jnp.dot(c, wuk_ref[...],
                             preferred_element_type=jnp.float32).astype(BF)
    acc_ref[:, D:] = jnp.dot(c, wuv_ref[...],
                             preferred_element_type=jnp.float32).astype(BF)

    def rs_step(h):
        j_send = (my_y - h) % N_Y
        return pltpu.make_async_remote_copy(
            src_ref=acc_ref.at[pl.ds(j_send * CHUNK, CHUNK), :],
            dst_ref=rs_recv_ref.at[h],
            send_sem=rs_send_sems.at[h],
            recv_sem=rs_recv_sems.at[h],
            device_id=right,
            device_id_type=pl.DeviceIdType.MESH,
        )

    rdma0 = rs_step(0)
    rdma0.start()

    q_ref[...] = jnp.dot(x, wq_ref[...],
                         preferred_element_type=jnp.float32).astype(BF)
    qr_ref[...] = jnp.dot(x, wqr_ref[...],
                          preferred_element_type=jnp.float32).astype(BF)
    kr_ref[...] = jnp.dot(x, wkr_ref[...],
                          preferred_element_type=jnp.float32).astype(BF)

    for h in range(N_Y - 1):
        rdma = rdma0 if h == 0 else rs_step(h)
        if h != 0:
            rdma.start()
        rdma.wait()
        row = ((my_y - h - 1) % N_Y) * CHUNK
        acc_ref[pl.ds(row, CHUNK), :] = (
            acc_ref[pl.ds(row, CHUNK), :].astype(jnp.float32)
            + rs_recv_ref[h].astype(jnp.float32)
        ).astype(BF)

    for h in range(N_Y - 1):
        j = (my_y + 1 - h) % N_Y
        rdma = pltpu.make_async_remote_copy(
            src_ref=acc_ref.at[pl.ds(j * CHUNK, CHUNK), :],
            dst_ref=acc_ref.at[pl.ds(j * CHUNK, CHUNK), :],
            send_sem=ag_send_sems.at[h],
            recv_sem=ag_recv_sems.at[h],
            device_id=right,
            device_id_type=pl.DeviceIdType.MESH,
        )
        rdma.start()
        rdma.wait()

    k_ref[...] = acc_ref[:, :D]
    v_ref[...] = acc_ref[:, D:]


def _attn_body(q_ref, k_ref, v_ref, qr_ref, kr_ref, o_ref):
    s = lax.dot_general(q_ref[...], k_ref[...], (((1,), (1,)), ((), ())),
                        preferred_element_type=jnp.float32)
    s = s + lax.dot_general(qr_ref[...], kr_ref[...], (((1,), (1,)), ((), ())),
                            preferred_element_type=jnp.float32)
    s = s * SCALE
    m = jnp.max(s, axis=1, keepdims=True)
    p = jnp.exp(s - m)
    p = p / jnp.sum(p, axis=1, keepdims=True)
    o_ref[...] = jnp.dot(p.astype(BF), v_ref[...],
                         preferred_element_type=jnp.float32).astype(BF)


def _proj_body(o_ref, wo_ref, out_ref):
    out_ref[...] = jnp.dot(o_ref[...], wo_ref[...],
                           preferred_element_type=jnp.float32)


def kernel(x, Wdkv, Wuk, Wuv, Wq, Wqr, Wkr, Wo):
    xb = x.reshape(S, D).astype(BF)
    vmem = pl.BlockSpec(memory_space=pltpu.VMEM)

    k, v, q, qr, kr = pl.pallas_call(
        _reduce_body,
        out_shape=[
            jax.ShapeDtypeStruct((S, D), BF),
            jax.ShapeDtypeStruct((S, D), BF),
            jax.ShapeDtypeStruct((S, D), BF),
            jax.ShapeDtypeStruct((S, H * DR), BF),
            jax.ShapeDtypeStruct((S, DR), BF),
        ],
        in_specs=[vmem] * 7,
        out_specs=[vmem] * 5,
        scratch_shapes=[
            pltpu.VMEM((S, KV), BF),
            pltpu.VMEM((N_Y - 1, CHUNK, KV), BF),
            pltpu.SemaphoreType.DMA((N_Y - 1,)),
            pltpu.SemaphoreType.DMA((N_Y - 1,)),
            pltpu.SemaphoreType.DMA((N_Y - 1,)),
            pltpu.SemaphoreType.DMA((N_Y - 1,)),
        ],
        compiler_params=pltpu.CompilerParams(collective_id=0),
    )(xb, Wdkv.astype(BF), Wuk.astype(BF), Wuv.astype(BF),
      Wq.astype(BF), Wqr.astype(BF), Wkr.astype(BF))

    o = pl.pallas_call(
        _attn_body,
        grid=(H,),
        out_shape=jax.ShapeDtypeStruct((S, D), BF),
        in_specs=[
            pl.BlockSpec((S, DH), lambda h: (0, h)),
            pl.BlockSpec((S, DH), lambda h: (0, h)),
            pl.BlockSpec((S, DH), lambda h: (0, h)),
            pl.BlockSpec((S, DR), lambda h: (0, h)),
            pl.BlockSpec((S, DR), lambda h: (0, 0)),
        ],
        out_specs=pl.BlockSpec((S, DH), lambda h: (0, h)),
    )(q, k, v, qr, kr)

    out = pl.pallas_call(
        _proj_body,
        out_shape=jax.ShapeDtypeStruct((S, D), jnp.float32),
        in_specs=[vmem, vmem],
        out_specs=vmem,
    )(o, Wo.astype(BF))
    return out.reshape(1, S, D)


# baseline (device time: 263283 ns/iter reference)
import jax
import jax.numpy as jnp
from jax import lax
from jax.experimental import pallas as pl
from jax.experimental.pallas import tpu as pltpu

N_Y = 4
S = 1024
D = 2048
H = 16
DH = 128
DR = 32
CHUNK = S // N_Y
KV = 2 * D
SCALE = (DH + DR) ** -0.5
BF = jnp.bfloat16


def _reduce_body(x_ref, wdkv_ref, wuk_ref, wuv_ref, wq_ref, wqr_ref, wkr_ref,
                 k_ref, v_ref, q_ref, qr_ref, kr_ref,
                 acc_ref, rs_recv_ref,
                 rs_send_sems, rs_recv_sems, ag_send_sems, ag_recv_sems):
    my_x = lax.axis_index("x")
    my_y = lax.axis_index("y")
    my_z = lax.axis_index("z")
    right = (my_x, (my_y + 1) % N_Y, my_z)
    left = (my_x, (my_y - 1) % N_Y, my_z)

    barrier = pltpu.get_barrier_semaphore()
    for nbr in (left, right):
        pl.semaphore_signal(barrier, inc=1, device_id=nbr,
                            device_id_type=pl.DeviceIdType.MESH)
    pl.semaphore_wait(barrier, 2)

    x = x_ref[...]
    c = jnp.dot(x, wdkv_ref[...], preferred_element_type=jnp.float32).astype(BF)
    acc_ref[:, :D] = jnp.dot(c, wuk_ref[...],
                             preferred_element_type=jnp.float32).astype(BF)
    acc_ref[:, D:] = jnp.dot(c, wuv_ref[...],
                             preferred_element_type=jnp.float32).astype(BF)

    def rs_step(h):
        j_send = (my_y - h) % N_Y
        return pltpu.make_async_remote_copy(
            src_ref=acc_ref.at[pl.ds(j_send * CHUNK, CHUNK), :],
            dst_ref=rs_recv_ref.at[h],
            send_sem=rs_send_sems.at[h],
            recv_sem=rs_recv_sems.at[h],
            device_id=right,
            device_id_type=pl.DeviceIdType.MESH,
        )

    rdma0 = rs_step(0)
    rdma0.start()

    q_ref[...] = jnp.dot(x, wq_ref[...],
                         preferred_element_type=jnp.float32).astype(BF)
    qr_full = jnp.dot(x, wqr_ref[...],
                      preferred_element_type=jnp.float32).astype(BF)
    qr_ref[...] = jnp.zeros_like(qr_ref)
    for hh in range(H):
        qr_ref[:, hh * DH:hh * DH + DR] = qr_full[:, hh * DR:(hh + 1) * DR]
    kr_ref[...] = jnp.zeros_like(kr_ref)
    kr_ref[:, :DR] = jnp.dot(x, wkr_ref[...],
                             preferred_element_type=jnp.float32).astype(BF)

    for h in range(N_Y - 1):
        rdma = rdma0 if h == 0 else rs_step(h)
        if h != 0:
            rdma.start()
        rdma.wait()
        row = ((my_y - h - 1) % N_Y) * CHUNK
        acc_ref[pl.ds(row, CHUNK), :] = (
            acc_ref[pl.ds(row, CHUNK), :].astype(jnp.float32)
            + rs_recv_ref[h].astype(jnp.float32)
        ).astype(BF)

    for h in range(N_Y - 1):
        j = (my_y + 1 - h) % N_Y
        rdma = pltpu.make_async_remote_copy(
            src_ref=acc_ref.at[pl.ds(j * CHUNK, CHUNK), :],
            dst_ref=acc_ref.at[pl.ds(j * CHUNK, CHUNK), :],
            send_sem=ag_send_sems.at[h],
            recv_sem=ag_recv_sems.at[h],
            device_id=right,
            device_id_type=pl.DeviceIdType.MESH,
        )
        rdma.start()
        rdma.wait()

    k_ref[...] = acc_ref[:, :D]
    v_ref[...] = acc_ref[:, D:]


def _attn_body(q_ref, k_ref, v_ref, qr_ref, kr_ref, o_ref):
    s = lax.dot_general(q_ref[...], k_ref[...], (((1,), (1,)), ((), ())),
                        preferred_element_type=jnp.float32)
    s = s + lax.dot_general(qr_ref[...], kr_ref[...], (((1,), (1,)), ((), ())),
                            preferred_element_type=jnp.float32)
    s = s * SCALE
    m = jnp.max(s, axis=1, keepdims=True)
    p = jnp.exp(s - m)
    p = p / jnp.sum(p, axis=1, keepdims=True)
    o_ref[...] = jnp.dot(p.astype(BF), v_ref[...],
                         preferred_element_type=jnp.float32).astype(BF)


def _proj_body(o_ref, wo_ref, out_ref):
    out_ref[...] = jnp.dot(o_ref[...], wo_ref[...],
                           preferred_element_type=jnp.float32)


def kernel(x, Wdkv, Wuk, Wuv, Wq, Wqr, Wkr, Wo):
    xb = x.reshape(S, D).astype(BF)
    vmem = pl.BlockSpec(memory_space=pltpu.VMEM)

    k, v, q, qr, kr = pl.pallas_call(
        _reduce_body,
        out_shape=[
            jax.ShapeDtypeStruct((S, D), BF),
            jax.ShapeDtypeStruct((S, D), BF),
            jax.ShapeDtypeStruct((S, D), BF),
            jax.ShapeDtypeStruct((S, H * DH), BF),
            jax.ShapeDtypeStruct((S, DH), BF),
        ],
        in_specs=[vmem] * 7,
        out_specs=[vmem] * 5,
        scratch_shapes=[
            pltpu.VMEM((S, KV), BF),
            pltpu.VMEM((N_Y - 1, CHUNK, KV), BF),
            pltpu.SemaphoreType.DMA((N_Y - 1,)),
            pltpu.SemaphoreType.DMA((N_Y - 1,)),
            pltpu.SemaphoreType.DMA((N_Y - 1,)),
            pltpu.SemaphoreType.DMA((N_Y - 1,)),
        ],
        compiler_params=pltpu.CompilerParams(collective_id=0),
    )(xb, Wdkv.astype(BF), Wuk.astype(BF), Wuv.astype(BF),
      Wq.astype(BF), Wqr.astype(BF), Wkr.astype(BF))

    o = pl.pallas_call(
        _attn_body,
        grid=(H,),
        out_shape=jax.ShapeDtypeStruct((S, D), BF),
        in_specs=[
            pl.BlockSpec((S, DH), lambda h: (0, h)),
            pl.BlockSpec((S, DH), lambda h: (0, h)),
            pl.BlockSpec((S, DH), lambda h: (0, h)),
            pl.BlockSpec((S, DH), lambda h: (0, h)),
            pl.BlockSpec((S, DH), lambda h: (0, 0)),
        ],
        out_specs=pl.BlockSpec((S, DH), lambda h: (0, h)),
    )(q, k, v, qr, kr)

    out = pl.pallas_call(
        _proj_body,
        out_shape=jax.ShapeDtypeStruct((S, D), jnp.float32),
        in_specs=[vmem, vmem],
        out_specs=vmem,
    )(o, Wo.astype(BF))
    return out.reshape(1, S, D)
